# Initial kernel scaffold; baseline (speedup 1.0000x reference)
import jax
import jax.numpy as jnp
from jax.experimental import pallas as pl


def kernel(seg_edge, seg_body, contrast_logits, contrast_target, confidence, target, gt_boundary, sem_gt):
    raise NotImplementedError("write your pallas kernel here")


# speedup vs baseline: 5.3622x; 5.3622x over previous
import numpy as np
import jax
import jax.numpy as jnp
from jax.experimental import pallas as pl


def _interp_matrix(n_in, n_out):
    xs = np.linspace(0.0, n_in - 1.0, n_out, dtype=np.float32)
    x0 = np.floor(xs).astype(np.int32)
    x1 = np.minimum(x0 + 1, n_in - 1)
    wx = (xs - x0.astype(np.float32)).astype(np.float32)
    W = np.zeros((n_out, n_in), dtype=np.float32)
    W[np.arange(n_out), x0] += 1.0 - wx
    W[np.arange(n_out), x1] += wx
    return W


def _make_body(num_classes, precision):
    def _body(seg_ref, conf_ref, gb_ref, sem_ref, wy_ref, wxt_ref,
              num_ref, den_ref):
        bi = pl.program_id(0)
        ci = pl.program_id(1)

        wy = wy_ref[...]
        wxt = wxt_ref[...]

        labels = sem_ref[0]
        vf = (gb_ref[0] == 0).astype(jnp.float32)

        dot = lambda a, b: jnp.dot(a, b, precision=precision,
                                   preferred_element_type=jnp.float32)

        conf_up = dot(dot(wy, conf_ref[0]), wxt)

        ups = [dot(dot(wy, seg_ref[0, c]), wxt) for c in range(num_classes)]
        m = ups[0]
        for c in range(1, num_classes):
            m = jnp.maximum(m, ups[c])
        s = jnp.exp(ups[0] - m)
        sel = jnp.where(labels == 0, ups[0], 0.0)
        for c in range(1, num_classes):
            s = s + jnp.exp(ups[c] - m)
            sel = sel + jnp.where(labels == c, ups[c], 0.0)
        lse = m + jnp.log(s)

        nll = lse - sel
        pnum = jnp.sum(nll * conf_up * vf).reshape(1, 1)
        pden = jnp.sum(vf).reshape(1, 1)

        first = jnp.logical_and(bi == 0, ci == 0)

        @pl.when(first)
        def _():
            num_ref[...] = pnum
            den_ref[...] = pden

        @pl.when(jnp.logical_not(first))
        def _():
            num_ref[...] = num_ref[...] + pnum
            den_ref[...] = den_ref[...] + pden

    return _body


def kernel(seg_edge, seg_body, contrast_logits, contrast_target,
           confidence, target, gt_boundary, sem_gt):
    b, nc, h, w = seg_body.shape
    hg, wg = sem_gt.shape[1], sem_gt.shape[2]
    chunk = 128
    nchunks = hg // chunk

    wy = jnp.asarray(_interp_matrix(h, hg))
    wxt = jnp.asarray(_interp_matrix(w, wg).T)
    gb = gt_boundary.astype(jnp.int32)

    num, den = pl.pallas_call(
        _make_body(nc, jax.lax.Precision.HIGHEST),
        grid=(b, nchunks),
        in_specs=[
            pl.BlockSpec((1, nc, h, w), lambda i, j: (i, 0, 0, 0)),
            pl.BlockSpec((1, h, w), lambda i, j: (i, 0, 0)),
            pl.BlockSpec((1, chunk, wg), lambda i, j: (i, j, 0)),
            pl.BlockSpec((1, chunk, wg), lambda i, j: (i, j, 0)),
            pl.BlockSpec((chunk, h), lambda i, j: (j, 0)),
            pl.BlockSpec((w, wg), lambda i, j: (0, 0)),
        ],
        out_specs=[
            pl.BlockSpec((1, 1), lambda i, j: (0, 0)),
            pl.BlockSpec((1, 1), lambda i, j: (0, 0)),
        ],
        out_shape=[
            jax.ShapeDtypeStruct((1, 1), jnp.float32),
            jax.ShapeDtypeStruct((1, 1), jnp.float32),
        ],
    )(seg_body, confidence, gb, sem_gt, wy, wxt)

    return num[0, 0] / jnp.maximum(den[0, 0], 1.0)


# speedup vs baseline: 11.2337x; 2.0950x over previous
import numpy as np
import jax
import jax.numpy as jnp
from jax.experimental import pallas as pl


def _interp_matrix(n_in, n_out):
    xs = np.linspace(0.0, n_in - 1.0, n_out, dtype=np.float32)
    x0 = np.floor(xs).astype(np.int32)
    x1 = np.minimum(x0 + 1, n_in - 1)
    wx = (xs - x0.astype(np.float32)).astype(np.float32)
    W = np.zeros((n_out, n_in), dtype=np.float32)
    W[np.arange(n_out), x0] += 1.0 - wx
    W[np.arange(n_out), x1] += wx
    return W


def _make_body(num_classes, precision):
    def _body(seg_ref, conf_ref, gb_ref, sem_ref, wy_ref, wxt_ref,
              num_ref, den_ref):
        bi = pl.program_id(0)
        ci = pl.program_id(1)

        wy = wy_ref[...]
        wxt = wxt_ref[...]

        labels = sem_ref[0]
        vf = (gb_ref[0] == 0).astype(jnp.float32)

        dot = lambda a, b: jnp.dot(a, b, precision=precision,
                                   preferred_element_type=jnp.float32)

        conf_up = dot(dot(wy, conf_ref[0]), wxt)

        ups = [dot(dot(wy, seg_ref[0, c]), wxt) for c in range(num_classes)]
        m = ups[0]
        for c in range(1, num_classes):
            m = jnp.maximum(m, ups[c])
        s = jnp.exp(ups[0] - m)
        sel = jnp.where(labels == 0, ups[0], 0.0)
        for c in range(1, num_classes):
            s = s + jnp.exp(ups[c] - m)
            sel = sel + jnp.where(labels == c, ups[c], 0.0)
        lse = m + jnp.log(s)

        nll = lse - sel
        pnum = jnp.sum(nll * conf_up * vf).reshape(1, 1)
        pden = jnp.sum(vf).reshape(1, 1)

        first = jnp.logical_and(bi == 0, ci == 0)

        @pl.when(first)
        def _():
            num_ref[...] = pnum
            den_ref[...] = pden

        @pl.when(jnp.logical_not(first))
        def _():
            num_ref[...] = num_ref[...] + pnum
            den_ref[...] = den_ref[...] + pden

    return _body


def kernel(seg_edge, seg_body, contrast_logits, contrast_target,
           confidence, target, gt_boundary, sem_gt):
    b, nc, h, w = seg_body.shape
    hg, wg = sem_gt.shape[1], sem_gt.shape[2]
    chunk = 128
    nchunks = hg // chunk

    wy = jnp.asarray(_interp_matrix(h, hg))
    wxt = jnp.asarray(_interp_matrix(w, wg).T)
    gb = gt_boundary.astype(jnp.int32)

    num, den = pl.pallas_call(
        _make_body(nc, jax.lax.Precision.DEFAULT),
        grid=(b, nchunks),
        in_specs=[
            pl.BlockSpec((1, nc, h, w), lambda i, j: (i, 0, 0, 0)),
            pl.BlockSpec((1, h, w), lambda i, j: (i, 0, 0)),
            pl.BlockSpec((1, chunk, wg), lambda i, j: (i, j, 0)),
            pl.BlockSpec((1, chunk, wg), lambda i, j: (i, j, 0)),
            pl.BlockSpec((chunk, h), lambda i, j: (j, 0)),
            pl.BlockSpec((w, wg), lambda i, j: (0, 0)),
        ],
        out_specs=[
            pl.BlockSpec((1, 1), lambda i, j: (0, 0)),
            pl.BlockSpec((1, 1), lambda i, j: (0, 0)),
        ],
        out_shape=[
            jax.ShapeDtypeStruct((1, 1), jnp.float32),
            jax.ShapeDtypeStruct((1, 1), jnp.float32),
        ],
    )(seg_body, confidence, gb, sem_gt, wy, wxt)

    return num[0, 0] / jnp.maximum(den[0, 0], 1.0)


# speedup vs baseline: 12.6494x; 1.1260x over previous
import numpy as np
import jax
import jax.numpy as jnp
from jax.experimental import pallas as pl


def _interp_matrix(n_in, n_out):
    xs = np.linspace(0.0, n_in - 1.0, n_out, dtype=np.float32)
    x0 = np.floor(xs).astype(np.int32)
    x1 = np.minimum(x0 + 1, n_in - 1)
    wx = (xs - x0.astype(np.float32)).astype(np.float32)
    W = np.zeros((n_out, n_in), dtype=np.float32)
    W[np.arange(n_out), x0] += 1.0 - wx
    W[np.arange(n_out), x1] += wx
    return W


def _make_body(num_classes, precision):
    def _body(seg_ref, conf_ref, gb_ref, sem_ref, wy_ref, wxt_ref,
              num_ref, den_ref):
        bi = pl.program_id(0)
        ci = pl.program_id(1)

        wy = wy_ref[...]
        wxt = wxt_ref[...]

        labels = sem_ref[0]
        vf = (gb_ref[0] == 0).astype(jnp.float32)

        dot = lambda a, b: jnp.dot(a, b, precision=precision,
                                   preferred_element_type=jnp.float32)

        conf_up = dot(dot(wy, conf_ref[0]), wxt)

        cmax = [seg_ref[0, c] for c in range(num_classes)]
        while len(cmax) > 1:
            nxt = [jnp.maximum(cmax[i], cmax[i + 1])
                   for i in range(0, len(cmax) - 1, 2)]
            if len(cmax) % 2:
                nxt.append(cmax[-1])
            cmax = nxt
        m = dot(dot(wy, cmax[0]), wxt)

        s = jnp.zeros_like(m)
        sel = jnp.zeros_like(m)
        for c in range(num_classes):
            up_c = dot(dot(wy, seg_ref[0, c]), wxt)
            s = s + jnp.exp(up_c - m)
            sel = sel + jnp.where(labels == c, up_c, 0.0)
        lse = m + jnp.log(s)

        nll = lse - sel
        pnum = jnp.sum(nll * conf_up * vf).reshape(1, 1)
        pden = jnp.sum(vf).reshape(1, 1)

        first = jnp.logical_and(bi == 0, ci == 0)

        @pl.when(first)
        def _():
            num_ref[...] = pnum
            den_ref[...] = pden

        @pl.when(jnp.logical_not(first))
        def _():
            num_ref[...] = num_ref[...] + pnum
            den_ref[...] = den_ref[...] + pden

    return _body


def kernel(seg_edge, seg_body, contrast_logits, contrast_target,
           confidence, target, gt_boundary, sem_gt):
    b, nc, h, w = seg_body.shape
    hg, wg = sem_gt.shape[1], sem_gt.shape[2]
    chunk = 128
    nchunks = hg // chunk

    wy = jnp.asarray(_interp_matrix(h, hg))
    wxt = jnp.asarray(_interp_matrix(w, wg).T)
    gb = gt_boundary.astype(jnp.int32)

    num, den = pl.pallas_call(
        _make_body(nc, jax.lax.Precision.DEFAULT),
        grid=(b, nchunks),
        in_specs=[
            pl.BlockSpec((1, nc, h, w), lambda i, j: (i, 0, 0, 0)),
            pl.BlockSpec((1, h, w), lambda i, j: (i, 0, 0)),
            pl.BlockSpec((1, chunk, wg), lambda i, j: (i, j, 0)),
            pl.BlockSpec((1, chunk, wg), lambda i, j: (i, j, 0)),
            pl.BlockSpec((chunk, h), lambda i, j: (j, 0)),
            pl.BlockSpec((w, wg), lambda i, j: (0, 0)),
        ],
        out_specs=[
            pl.BlockSpec((1, 1), lambda i, j: (0, 0)),
            pl.BlockSpec((1, 1), lambda i, j: (0, 0)),
        ],
        out_shape=[
            jax.ShapeDtypeStruct((1, 1), jnp.float32),
            jax.ShapeDtypeStruct((1, 1), jnp.float32),
        ],
    )(seg_body, confidence, gb, sem_gt, wy, wxt)

    return num[0, 0] / jnp.maximum(den[0, 0], 1.0)


# speedup vs baseline: 18.2310x; 1.4413x over previous
import numpy as np
import jax
import jax.numpy as jnp
from jax.experimental import pallas as pl


def _interp_matrix(n_in, n_out):
    xs = np.linspace(0.0, n_in - 1.0, n_out, dtype=np.float32)
    x0 = np.floor(xs).astype(np.int32)
    x1 = np.minimum(x0 + 1, n_in - 1)
    wx = (xs - x0.astype(np.float32)).astype(np.float32)
    W = np.zeros((n_out, n_in), dtype=np.float32)
    W[np.arange(n_out), x0] += 1.0 - wx
    W[np.arange(n_out), x1] += wx
    return W


def _make_body(num_classes, precision):
    def _body(seg_ref, conf_ref, gb_ref, sem_ref, wy_ref, wxt_ref,
              num_ref, den_ref):
        bi = pl.program_id(0)
        ci = pl.program_id(1)

        wy = wy_ref[...]
        wxt = wxt_ref[...]

        labels = sem_ref[0]
        vf = (gb_ref[0] == 0).astype(jnp.float32)

        dot = lambda a, b: jnp.dot(a, b, precision=precision,
                                   preferred_element_type=jnp.float32)

        conf_up = dot(dot(wy, conf_ref[0]), wxt)

        cmax = [seg_ref[0, c] for c in range(num_classes)]
        while len(cmax) > 1:
            nxt = [jnp.maximum(cmax[i], cmax[i + 1])
                   for i in range(0, len(cmax) - 1, 2)]
            if len(cmax) % 2:
                nxt.append(cmax[-1])
            cmax = nxt
        m = dot(dot(wy, cmax[0]), wxt)

        s = jnp.zeros_like(m)
        sel = jnp.zeros_like(m)
        for c in range(num_classes):
            up_c = dot(dot(wy, seg_ref[0, c]), wxt)
            s = s + jnp.exp(up_c - m)
            sel = sel + jnp.where(labels == c, up_c, 0.0)
        lse = m + jnp.log(s)

        nll = lse - sel
        pnum = jnp.sum(nll * conf_up * vf).reshape(1, 1)
        pden = jnp.sum(vf).reshape(1, 1)

        first = jnp.logical_and(bi == 0, ci == 0)

        @pl.when(first)
        def _():
            num_ref[...] = pnum
            den_ref[...] = pden

        @pl.when(jnp.logical_not(first))
        def _():
            num_ref[...] = num_ref[...] + pnum
            den_ref[...] = den_ref[...] + pden

    return _body


def kernel(seg_edge, seg_body, contrast_logits, contrast_target,
           confidence, target, gt_boundary, sem_gt):
    b, nc, h, w = seg_body.shape
    hg, wg = sem_gt.shape[1], sem_gt.shape[2]
    chunk = 256
    nchunks = hg // chunk

    wy = jnp.asarray(_interp_matrix(h, hg))
    wxt = jnp.asarray(_interp_matrix(w, wg).T)
    gb = gt_boundary.astype(jnp.int32)

    num, den = pl.pallas_call(
        _make_body(nc, jax.lax.Precision.DEFAULT),
        grid=(b, nchunks),
        in_specs=[
            pl.BlockSpec((1, nc, h, w), lambda i, j: (i, 0, 0, 0)),
            pl.BlockSpec((1, h, w), lambda i, j: (i, 0, 0)),
            pl.BlockSpec((1, chunk, wg), lambda i, j: (i, j, 0)),
            pl.BlockSpec((1, chunk, wg), lambda i, j: (i, j, 0)),
            pl.BlockSpec((chunk, h), lambda i, j: (j, 0)),
            pl.BlockSpec((w, wg), lambda i, j: (0, 0)),
        ],
        out_specs=[
            pl.BlockSpec((1, 1), lambda i, j: (0, 0)),
            pl.BlockSpec((1, 1), lambda i, j: (0, 0)),
        ],
        out_shape=[
            jax.ShapeDtypeStruct((1, 1), jnp.float32),
            jax.ShapeDtypeStruct((1, 1), jnp.float32),
        ],
    )(seg_body, confidence, gb, sem_gt, wy, wxt)

    return num[0, 0] / jnp.maximum(den[0, 0], 1.0)


# speedup vs baseline: 20.3937x; 1.1186x over previous
import numpy as np
import jax
import jax.numpy as jnp
from jax.experimental import pallas as pl


def _interp_matrix(n_in, n_out):
    xs = np.linspace(0.0, n_in - 1.0, n_out, dtype=np.float32)
    x0 = np.floor(xs).astype(np.int32)
    x1 = np.minimum(x0 + 1, n_in - 1)
    wx = (xs - x0.astype(np.float32)).astype(np.float32)
    W = np.zeros((n_out, n_in), dtype=np.float32)
    W[np.arange(n_out), x0] += 1.0 - wx
    W[np.arange(n_out), x1] += wx
    return W


def _make_body(num_classes, precision):
    def _body(seg_ref, conf_ref, gb_ref, sem_ref, wy_ref, wxt_ref,
              num_ref, den_ref):
        bi = pl.program_id(0)
        ci = pl.program_id(1)

        wy = wy_ref[...]
        wxt = wxt_ref[...]

        labels = sem_ref[0]
        vf = (gb_ref[0] == 0).astype(jnp.float32)

        dot = lambda a, b: jnp.dot(a, b, precision=precision,
                                   preferred_element_type=jnp.float32)

        conf_up = dot(dot(wy, conf_ref[0]), wxt)

        cmax = [seg_ref[0, c] for c in range(num_classes)]
        while len(cmax) > 1:
            nxt = [jnp.maximum(cmax[i], cmax[i + 1])
                   for i in range(0, len(cmax) - 1, 2)]
            if len(cmax) % 2:
                nxt.append(cmax[-1])
            cmax = nxt
        m = dot(dot(wy, cmax[0]), wxt)

        s = jnp.zeros_like(m)
        sel = jnp.zeros_like(m)
        for c in range(num_classes):
            up_c = dot(dot(wy, seg_ref[0, c]), wxt)
            s = s + jnp.exp(up_c - m)
            sel = sel + jnp.where(labels == c, up_c, 0.0)
        lse = m + jnp.log(s)

        nll = lse - sel
        pnum = jnp.sum(nll * conf_up * vf).reshape(1, 1)
        pden = jnp.sum(vf).reshape(1, 1)

        first = jnp.logical_and(bi == 0, ci == 0)

        @pl.when(first)
        def _():
            num_ref[...] = pnum
            den_ref[...] = pden

        @pl.when(jnp.logical_not(first))
        def _():
            num_ref[...] = num_ref[...] + pnum
            den_ref[...] = den_ref[...] + pden

    return _body


def kernel(seg_edge, seg_body, contrast_logits, contrast_target,
           confidence, target, gt_boundary, sem_gt):
    b, nc, h, w = seg_body.shape
    hg, wg = sem_gt.shape[1], sem_gt.shape[2]
    chunk = 512
    nchunks = hg // chunk

    wy = jnp.asarray(_interp_matrix(h, hg))
    wxt = jnp.asarray(_interp_matrix(w, wg).T)
    gb = gt_boundary.astype(jnp.int32)

    num, den = pl.pallas_call(
        _make_body(nc, jax.lax.Precision.DEFAULT),
        grid=(b, nchunks),
        in_specs=[
            pl.BlockSpec((1, nc, h, w), lambda i, j: (i, 0, 0, 0)),
            pl.BlockSpec((1, h, w), lambda i, j: (i, 0, 0)),
            pl.BlockSpec((1, chunk, wg), lambda i, j: (i, j, 0)),
            pl.BlockSpec((1, chunk, wg), lambda i, j: (i, j, 0)),
            pl.BlockSpec((chunk, h), lambda i, j: (j, 0)),
            pl.BlockSpec((w, wg), lambda i, j: (0, 0)),
        ],
        out_specs=[
            pl.BlockSpec((1, 1), lambda i, j: (0, 0)),
            pl.BlockSpec((1, 1), lambda i, j: (0, 0)),
        ],
        out_shape=[
            jax.ShapeDtypeStruct((1, 1), jnp.float32),
            jax.ShapeDtypeStruct((1, 1), jnp.float32),
        ],
    )(seg_body, confidence, gb, sem_gt, wy, wxt)

    return num[0, 0] / jnp.maximum(den[0, 0], 1.0)
